# baseline (speedup 1.0000x reference)
import functools

import numpy as np
import jax
import jax.numpy as jnp
from jax import lax
from jax.experimental import pallas as pl
from jax.experimental.pallas import tpu as pltpu
from jax.experimental.pallas import tpu_sc as plsc

_B = 4096
_NN = 200
_K = 10
_D = 64
_W1 = 1e-06
_W2 = 1.0
_W3 = 1e-06
_W4 = 1.0
_NEG_WEIGHT = 300.0
_GAMMA = 1e-4
_LAMBDA = 2.75

_info = plsc.get_sparse_core_info()
_NC, _NS = _info.num_cores, _info.num_subcores
_NW = _NC * _NS
_NB = _B // _NW
_NCH = 40
_CPB = _NN // _NCH
_NCHUNKS = _NB * _CPB
_IIC = 8

def _lane_set(acc, lane, jj, s):
    return jnp.where(lane == jj, s, 0.0 if acc is None else acc)


def _load4(buf, row):
    return [buf[row, pl.ds(t * 16, 16)] for t in range(4)]


def _dot64(buf, row, u4, lane):
    acc = buf[row, pl.ds(0, 16)] * u4[0]
    for t in range(1, 4):
        acc = acc + buf[row, pl.ds(t * 16, 16)] * u4[t]
    for sh in (8, 4, 2, 1):
        idx = jnp.bitwise_xor(lane, sh)
        acc = acc + acc.at[idx].get(mode="promise_in_bounds")
    return acc


def _sc_body(users_h, pos_h, negf_h, uemb_h, iemb_h, buD_h, biD_h, nbr_h,
             cons_h,
             possc_o, negsc_o, iisc_o, buo_o, bpo_o, bnego_o, simo_o,
             uidx_v, pidx_v, nidx_v, u_v, p_v,
             nb0_v, nb1_v, nb2_v, nb3_v, ii0_v, ii1_v, ii2_v, ii3_v,
             nbr_v, sim_v, bu_v, bp_v, bneg_v, possc_v, negsc_v, iisc_v,
             sem_u, sem_p, sem_bu, sem_bp, sem_nbr, sem_sim, sem_bneg,
             sem_n0, sem_n1, sem_n2, sem_n3, sem_i0, sem_i1, sem_i2, sem_i3):
    nbufs = ((nb0_v, sem_n0), (nb1_v, sem_n1), (nb2_v, sem_n2),
             (nb3_v, sem_n3))
    iibufs = ((ii0_v, sem_i0), (ii1_v, sem_i1), (ii2_v, sem_i2),
              (ii3_v, sem_i3))
    wid = lax.axis_index("s") * _NC + lax.axis_index("c")
    base = wid * _NB

    pltpu.sync_copy(users_h.at[pl.ds(base, _NB)], uidx_v)
    pltpu.sync_copy(pos_h.at[pl.ds(base, _NB)], pidx_v)
    pltpu.sync_copy(negf_h.at[pl.ds(base * _NN, _NB * _NN)], nidx_v)

    cu = pltpu.make_async_copy(uemb_h.at[uidx_v], u_v, sem_u)
    cu.start()
    cp = pltpu.make_async_copy(iemb_h.at[pidx_v], p_v, sem_p)
    cp.start()
    cbu = pltpu.make_async_copy(buD_h.at[uidx_v], bu_v, sem_bu)
    cbu.start()
    cbp = pltpu.make_async_copy(biD_h.at[pidx_v], bp_v, sem_bp)
    cbp.start()
    cnbr = pltpu.make_async_copy(nbr_h.at[pidx_v], nbr_v, sem_nbr)
    cnbr.start()
    csim = pltpu.make_async_copy(cons_h.at[pidx_v], sim_v, sem_sim)
    csim.start()

    def _fire_neg(c, buf, sem):
        idx = nidx_v.at[pl.ds(c * _NCH, _NCH)]
        pltpu.make_async_copy(iemb_h.at[idx], buf, sem).start()
        pltpu.make_async_copy(biD_h.at[idx],
                              bneg_v.at[pl.ds(c * _NCH, _NCH)],
                              sem_bneg).start()

    for par in range(4):
        _fire_neg(par, nbufs[par][0], nbufs[par][1])

    cu.wait()
    cp.wait()

    lane = lax.iota(jnp.int32, 16)

    def pos_body(gb, carry):
        b0 = gb * 16
        acc = None
        for jj in range(16):
            b = b0 + jj
            u4 = _load4(u_v, b)
            acc = _lane_set(acc, lane, jj, _dot64(p_v, b, u4, lane))
        possc_v[pl.ds(b0, 16)] = acc
        return carry
    lax.fori_loop(0, _NB // 16, pos_body, 0)

    def neg_body(g, carry):
        for par in range(4):
            buf, sem = nbufs[par]
            c = 4 * g + par
            idx = nidx_v.at[pl.ds(c * _NCH, _NCH)]
            pltpu.make_async_copy(iemb_h.at[idx], buf, sem).wait()
            pltpu.make_async_copy(biD_h.at[idx],
                                  bneg_v.at[pl.ds(c * _NCH, _NCH)],
                                  sem_bneg).wait()
            b = c // _CPB
            sbase = b * _NN + (c % _CPB) * _NCH
            u4 = _load4(u_v, b)
            for g0 in (0, 16, 24):
                acc = None
                for jj in range(16):
                    acc = _lane_set(acc, lane, jj,
                                    _dot64(buf, g0 + jj, u4, lane))
                negsc_v[pl.ds(sbase + g0, 16)] = acc

            nc = c + 4

            @pl.when(nc < _NCHUNKS)
            def _():
                _fire_neg(nc, buf, sem)
        return carry
    lax.fori_loop(0, _NCHUNKS // 4, neg_body, 0)

    cnbr.wait()

    def _fire_ii(b, buf, sem):
        pltpu.make_async_copy(iemb_h.at[nbr_v.at[b]], buf, sem).start()

    for par in range(4):
        _fire_ii(par, iibufs[par][0], iibufs[par][1])

    def ii_body(g, carry):
        for par in range(4):
            buf, sem = iibufs[par]
            b = 4 * g + par
            pltpu.make_async_copy(iemb_h.at[nbr_v.at[b]], buf, sem).wait()
            u4 = _load4(u_v, b)
            acc = None
            for j in range(_K):
                acc = _lane_set(acc, lane, j, _dot64(buf, j, u4, lane))
            acc = jnp.where(lane < _K, acc, 0.0)
            iisc_v[pl.ds(b * 16, 16)] = acc

            nb = b + 4

            @pl.when(nb < _NB)
            def _():
                _fire_ii(nb, buf, sem)
        return carry
    lax.fori_loop(0, _NB // 4, ii_body, 0)

    cbu.wait()
    cbp.wait()
    csim.wait()

    pltpu.sync_copy(possc_v, possc_o.at[pl.ds(base, _NB)])
    pltpu.sync_copy(negsc_v, negsc_o.at[pl.ds(base * _NN, _NB * _NN)])
    pltpu.sync_copy(iisc_v, iisc_o.at[pl.ds(base * 16, _NB * 16)])
    pltpu.sync_copy(bu_v, buo_o.at[pl.ds(base, _NB)])
    pltpu.sync_copy(bp_v, bpo_o.at[pl.ds(base, _NB)])
    pltpu.sync_copy(bneg_v, bnego_o.at[pl.ds(base * _NN, _NB * _NN)])
    pltpu.sync_copy(sim_v, simo_o.at[pl.ds(base, _NB)])


def _sc_gather_scores(users, pos_items, neg_flat, user_embeds, item_embeds,
                      beta_uD, beta_iD, ii_neighbor_mat, ii_constraint_mat):
    mesh = plsc.VectorSubcoreMesh(core_axis_name="c", subcore_axis_name="s")
    f32 = jnp.float32
    out_type = [
        jax.ShapeDtypeStruct((_B,), f32),
        jax.ShapeDtypeStruct((_B * _NN,), f32),
        jax.ShapeDtypeStruct((_B * 16,), f32),
        jax.ShapeDtypeStruct((_B,), f32),
        jax.ShapeDtypeStruct((_B,), f32),
        jax.ShapeDtypeStruct((_B * _NN,), f32),
        jax.ShapeDtypeStruct((_B, 16), f32),
    ]
    scratch_types = [
        pltpu.VMEM((_NB,), jnp.int32),
        pltpu.VMEM((_NB,), jnp.int32),
        pltpu.VMEM((_NB * _NN,), jnp.int32),
        pltpu.VMEM((_NB, _D), f32),
        pltpu.VMEM((_NB, _D), f32),
        pltpu.VMEM((_NCH, _D), f32),
        pltpu.VMEM((_NCH, _D), f32),
        pltpu.VMEM((_NCH, _D), f32),
        pltpu.VMEM((_NCH, _D), f32),
        pltpu.VMEM((16, _D), f32),
        pltpu.VMEM((16, _D), f32),
        pltpu.VMEM((16, _D), f32),
        pltpu.VMEM((16, _D), f32),
        pltpu.VMEM((_NB, 16), jnp.int32),
        pltpu.VMEM((_NB, 16), f32),
        pltpu.VMEM((_NB,), f32),
        pltpu.VMEM((_NB,), f32),
        pltpu.VMEM((_NB * _NN,), f32),
        pltpu.VMEM((_NB,), f32),
        pltpu.VMEM((_NB * _NN,), f32),
        pltpu.VMEM((_NB * 16,), f32),
    ] + [pltpu.SemaphoreType.DMA] * 15
    run = pl.kernel(_sc_body, mesh=mesh, out_type=out_type,
                    scratch_types=scratch_types,
                    compiler_params=pltpu.CompilerParams(
                        needs_layout_passes=False,
                        use_tc_tiling_on_sc=False))
    return run(users, pos_items, neg_flat, user_embeds, item_embeds,
               beta_uD, beta_iD, ii_neighbor_mat, ii_constraint_mat)


_RBLK = 2000
_GSTEPS = 100000 // _RBLK


def _tc_body(possc_r, negsc_r, iisc_r, bu_r, bp_r, bneg_r, sim_r, ue_r, ie_r,
             out_r):
    i = pl.program_id(0)

    @pl.when(i == 0)
    def _():
        bu = bu_r[...]
        pos_w = _W1 + _W2 * bu * bp_r[...]
        pos_loss = jnp.sum(pos_w * jax.nn.softplus(-possc_r[...]))
        neg_w = _W3 + _W4 * bu[:, None] * bneg_r[...]
        neg_loss = jnp.sum(neg_w * jax.nn.softplus(negsc_r[...]))
        ii_loss = jnp.sum(sim_r[...][:, :_K]
                          * jax.nn.softplus(-iisc_r[...][:, :_K]))
        out_r[0] = (pos_loss + neg_loss * (_NEG_WEIGHT / _NN)
                    + ii_loss * _LAMBDA)

    nrm = jnp.sum(ue_r[...] * ue_r[...]) + jnp.sum(ie_r[...] * ie_r[...])
    out_r[0] += (0.5 * _GAMMA) * nrm


def _tc_finish(possc, negsc, iisc, bu, bp, bneg, sim, user_embeds,
               item_embeds):
    full = lambda arr: pl.BlockSpec(arr.shape, lambda i: (0,) * arr.ndim)
    grid_spec = pl.GridSpec(
        grid=(_GSTEPS,),
        in_specs=[
            full(possc), full(negsc), full(iisc), full(bu), full(bp),
            full(bneg), full(sim),
            pl.BlockSpec((_RBLK, _D), lambda i: (i, 0)),
            pl.BlockSpec((_RBLK, _D), lambda i: (i, 0)),
        ],
        out_specs=pl.BlockSpec(memory_space=pltpu.SMEM),
    )
    out = pl.pallas_call(
        _tc_body,
        grid_spec=grid_spec,
        out_shape=jax.ShapeDtypeStruct((1,), jnp.float32),
        compiler_params=pltpu.CompilerParams(
            dimension_semantics=("arbitrary",)),
    )(possc, negsc, iisc, bu, bp, bneg, sim, user_embeds, item_embeds)
    return out[0]


def kernel(users, pos_items, neg_items, user_embeds, item_embeds, beta_uD,
           beta_iD, ii_neighbor_mat, ii_constraint_mat):
    neg_flat = neg_items.reshape(-1)
    nbr_pad = jnp.pad(ii_neighbor_mat, ((0, 0), (0, 16 - _K)))
    cons_pad = jnp.pad(ii_constraint_mat, ((0, 0), (0, 16 - _K)))
    possc, negsc, iisc, bu, bp, bneg, sim = _sc_gather_scores(
        users, pos_items, neg_flat, user_embeds, item_embeds, beta_uD,
        beta_iD, nbr_pad, cons_pad)
    return _tc_finish(possc, negsc.reshape(_B, _NN), iisc.reshape(_B, 16),
                      bu, bp, bneg.reshape(_B, _NN), sim, user_embeds,
                      item_embeds)


# speedup vs baseline: 1.0325x; 1.0325x over previous
import functools

import numpy as np
import jax
import jax.numpy as jnp
from jax import lax
from jax.experimental import pallas as pl
from jax.experimental.pallas import tpu as pltpu
from jax.experimental.pallas import tpu_sc as plsc

_B = 4096
_NN = 200
_K = 10
_D = 64
_W1 = 1e-06
_W2 = 1.0
_W3 = 1e-06
_W4 = 1.0
_NEG_WEIGHT = 300.0
_GAMMA = 1e-4
_LAMBDA = 2.75

_info = plsc.get_sparse_core_info()
_NC, _NS = _info.num_cores, _info.num_subcores
_NW = _NC * _NS
_NB = _B // _NW
_NCH = 128
_NCHUNKS = _NB * _NN // _NCH
_IIC = 8

def _lane_set(acc, lane, jj, s):
    return jnp.where(lane == jj, s, 0.0 if acc is None else acc)


def _load4(buf, row):
    return [buf[row, pl.ds(t * 16, 16)] for t in range(4)]


def _dot64(buf, row, u4, lane):
    acc = buf[row, pl.ds(0, 16)] * u4[0]
    for t in range(1, 4):
        acc = acc + buf[row, pl.ds(t * 16, 16)] * u4[t]
    for sh in (8, 4, 2, 1):
        idx = jnp.bitwise_xor(lane, sh)
        acc = acc + acc.at[idx].get(mode="promise_in_bounds")
    return acc


def _sc_body(users_h, pos_h, negf_h, uemb_h, iemb_h, buD_h, biD_h, nbr_h,
             cons_h,
             possc_o, negsc_o, iisc_o, buo_o, bpo_o, bnego_o, simo_o,
             uidx_v, pidx_v, nidx_v, u_v, p_v,
             nb0_v, nb1_v, ii0_v, ii1_v, ii2_v, ii3_v,
             nbr_v, sim_v, bu_v, bp_v, bneg_v, possc_v, negsc_v, iisc_v,
             sem_u, sem_p, sem_bu, sem_bp, sem_nbr, sem_sim, sem_bneg,
             sem_n0, sem_n1, sem_i0, sem_i1, sem_i2, sem_i3):
    nbufs = ((nb0_v, sem_n0), (nb1_v, sem_n1))
    iibufs = ((ii0_v, sem_i0), (ii1_v, sem_i1), (ii2_v, sem_i2),
              (ii3_v, sem_i3))
    wid = lax.axis_index("s") * _NC + lax.axis_index("c")
    base = wid * _NB

    pltpu.sync_copy(users_h.at[pl.ds(base, _NB)], uidx_v)
    pltpu.sync_copy(pos_h.at[pl.ds(base, _NB)], pidx_v)
    pltpu.sync_copy(negf_h.at[pl.ds(base * _NN, _NB * _NN)], nidx_v)

    cu = pltpu.make_async_copy(uemb_h.at[uidx_v], u_v, sem_u)
    cu.start()
    cp = pltpu.make_async_copy(iemb_h.at[pidx_v], p_v, sem_p)
    cp.start()
    cbu = pltpu.make_async_copy(buD_h.at[uidx_v], bu_v, sem_bu)
    cbu.start()
    cbp = pltpu.make_async_copy(biD_h.at[pidx_v], bp_v, sem_bp)
    cbp.start()
    cnbr = pltpu.make_async_copy(nbr_h.at[pidx_v], nbr_v, sem_nbr)
    cnbr.start()
    csim = pltpu.make_async_copy(cons_h.at[pidx_v], sim_v, sem_sim)
    csim.start()

    def _fire_neg(c, buf, sem):
        idx = nidx_v.at[pl.ds(c * _NCH, _NCH)]
        pltpu.make_async_copy(iemb_h.at[idx], buf, sem).start()
        pltpu.make_async_copy(biD_h.at[idx],
                              bneg_v.at[pl.ds(c * _NCH, _NCH)],
                              sem_bneg).start()

    for par in range(2):
        _fire_neg(par, nbufs[par][0], nbufs[par][1])

    cu.wait()
    cp.wait()

    lane = lax.iota(jnp.int32, 16)

    def pos_body(gb, carry):
        b0 = gb * 16
        acc = None
        for jj in range(16):
            b = b0 + jj
            u4 = _load4(u_v, b)
            acc = _lane_set(acc, lane, jj, _dot64(p_v, b, u4, lane))
        possc_v[pl.ds(b0, 16)] = acc
        return carry
    lax.fori_loop(0, _NB // 16, pos_body, 0)

    def neg_body(g, carry):
        for par in range(2):
            buf, sem = nbufs[par]
            c = 2 * g + par
            idx = nidx_v.at[pl.ds(c * _NCH, _NCH)]
            pltpu.make_async_copy(iemb_h.at[idx], buf, sem).wait()
            pltpu.make_async_copy(biD_h.at[idx],
                                  bneg_v.at[pl.ds(c * _NCH, _NCH)],
                                  sem_bneg).wait()
            cbase = c * _NCH

            def grp(k, kc):
                r0 = k * 16
                acc = None
                for jj in range(16):
                    b = (cbase + r0 + jj) // _NN
                    u4 = _load4(u_v, b)
                    acc = _lane_set(acc, lane, jj,
                                    _dot64(buf, r0 + jj, u4, lane))
                negsc_v[pl.ds(cbase + r0, 16)] = acc
                return kc
            lax.fori_loop(0, _NCH // 16, grp, 0)

            nc = c + 2

            @pl.when(nc < _NCHUNKS)
            def _():
                _fire_neg(nc, buf, sem)
        return carry
    lax.fori_loop(0, _NCHUNKS // 2, neg_body, 0)

    cnbr.wait()

    def _fire_ii(b, buf, sem):
        pltpu.make_async_copy(iemb_h.at[nbr_v.at[b]], buf, sem).start()

    for par in range(4):
        _fire_ii(par, iibufs[par][0], iibufs[par][1])

    def ii_body(g, carry):
        for par in range(4):
            buf, sem = iibufs[par]
            b = 4 * g + par
            pltpu.make_async_copy(iemb_h.at[nbr_v.at[b]], buf, sem).wait()
            u4 = _load4(u_v, b)
            acc = None
            for j in range(_K):
                acc = _lane_set(acc, lane, j, _dot64(buf, j, u4, lane))
            acc = jnp.where(lane < _K, acc, 0.0)
            iisc_v[pl.ds(b * 16, 16)] = acc

            nb = b + 4

            @pl.when(nb < _NB)
            def _():
                _fire_ii(nb, buf, sem)
        return carry
    lax.fori_loop(0, _NB // 4, ii_body, 0)

    cbu.wait()
    cbp.wait()
    csim.wait()

    pltpu.sync_copy(possc_v, possc_o.at[pl.ds(base, _NB)])
    pltpu.sync_copy(negsc_v, negsc_o.at[pl.ds(base * _NN, _NB * _NN)])
    pltpu.sync_copy(iisc_v, iisc_o.at[pl.ds(base * 16, _NB * 16)])
    pltpu.sync_copy(bu_v, buo_o.at[pl.ds(base, _NB)])
    pltpu.sync_copy(bp_v, bpo_o.at[pl.ds(base, _NB)])
    pltpu.sync_copy(bneg_v, bnego_o.at[pl.ds(base * _NN, _NB * _NN)])
    pltpu.sync_copy(sim_v, simo_o.at[pl.ds(base, _NB)])


def _sc_gather_scores(users, pos_items, neg_flat, user_embeds, item_embeds,
                      beta_uD, beta_iD, ii_neighbor_mat, ii_constraint_mat):
    mesh = plsc.VectorSubcoreMesh(core_axis_name="c", subcore_axis_name="s")
    f32 = jnp.float32
    out_type = [
        jax.ShapeDtypeStruct((_B,), f32),
        jax.ShapeDtypeStruct((_B * _NN,), f32),
        jax.ShapeDtypeStruct((_B * 16,), f32),
        jax.ShapeDtypeStruct((_B,), f32),
        jax.ShapeDtypeStruct((_B,), f32),
        jax.ShapeDtypeStruct((_B * _NN,), f32),
        jax.ShapeDtypeStruct((_B, 16), f32),
    ]
    scratch_types = [
        pltpu.VMEM((_NB,), jnp.int32),
        pltpu.VMEM((_NB,), jnp.int32),
        pltpu.VMEM((_NB * _NN,), jnp.int32),
        pltpu.VMEM((_NB, _D), f32),
        pltpu.VMEM((_NB, _D), f32),
        pltpu.VMEM((_NCH, _D), f32),
        pltpu.VMEM((_NCH, _D), f32),
        pltpu.VMEM((16, _D), f32),
        pltpu.VMEM((16, _D), f32),
        pltpu.VMEM((16, _D), f32),
        pltpu.VMEM((16, _D), f32),
        pltpu.VMEM((_NB, 16), jnp.int32),
        pltpu.VMEM((_NB, 16), f32),
        pltpu.VMEM((_NB,), f32),
        pltpu.VMEM((_NB,), f32),
        pltpu.VMEM((_NB * _NN,), f32),
        pltpu.VMEM((_NB,), f32),
        pltpu.VMEM((_NB * _NN,), f32),
        pltpu.VMEM((_NB * 16,), f32),
    ] + [pltpu.SemaphoreType.DMA] * 13
    run = pl.kernel(_sc_body, mesh=mesh, out_type=out_type,
                    scratch_types=scratch_types,
                    compiler_params=pltpu.CompilerParams(
                        needs_layout_passes=False,
                        use_tc_tiling_on_sc=False))
    return run(users, pos_items, neg_flat, user_embeds, item_embeds,
               beta_uD, beta_iD, ii_neighbor_mat, ii_constraint_mat)


_RBLK = 2000
_GSTEPS = 100000 // _RBLK


def _tc_body(possc_r, negsc_r, iisc_r, bu_r, bp_r, bneg_r, sim_r, ue_r, ie_r,
             out_r):
    i = pl.program_id(0)

    @pl.when(i == 0)
    def _():
        bu = bu_r[...]
        pos_w = _W1 + _W2 * bu * bp_r[...]
        pos_loss = jnp.sum(pos_w * jax.nn.softplus(-possc_r[...]))
        neg_w = _W3 + _W4 * bu[:, None] * bneg_r[...]
        neg_loss = jnp.sum(neg_w * jax.nn.softplus(negsc_r[...]))
        ii_loss = jnp.sum(sim_r[...][:, :_K]
                          * jax.nn.softplus(-iisc_r[...][:, :_K]))
        out_r[0] = (pos_loss + neg_loss * (_NEG_WEIGHT / _NN)
                    + ii_loss * _LAMBDA)

    nrm = jnp.sum(ue_r[...] * ue_r[...]) + jnp.sum(ie_r[...] * ie_r[...])
    out_r[0] += (0.5 * _GAMMA) * nrm


def _tc_finish(possc, negsc, iisc, bu, bp, bneg, sim, user_embeds,
               item_embeds):
    full = lambda arr: pl.BlockSpec(arr.shape, lambda i: (0,) * arr.ndim)
    grid_spec = pl.GridSpec(
        grid=(_GSTEPS,),
        in_specs=[
            full(possc), full(negsc), full(iisc), full(bu), full(bp),
            full(bneg), full(sim),
            pl.BlockSpec((_RBLK, _D), lambda i: (i, 0)),
            pl.BlockSpec((_RBLK, _D), lambda i: (i, 0)),
        ],
        out_specs=pl.BlockSpec(memory_space=pltpu.SMEM),
    )
    out = pl.pallas_call(
        _tc_body,
        grid_spec=grid_spec,
        out_shape=jax.ShapeDtypeStruct((1,), jnp.float32),
        compiler_params=pltpu.CompilerParams(
            dimension_semantics=("arbitrary",)),
    )(possc, negsc, iisc, bu, bp, bneg, sim, user_embeds, item_embeds)
    return out[0]


def kernel(users, pos_items, neg_items, user_embeds, item_embeds, beta_uD,
           beta_iD, ii_neighbor_mat, ii_constraint_mat):
    neg_flat = neg_items.reshape(-1)
    nbr_pad = jnp.pad(ii_neighbor_mat, ((0, 0), (0, 16 - _K)))
    cons_pad = jnp.pad(ii_constraint_mat, ((0, 0), (0, 16 - _K)))
    possc, negsc, iisc, bu, bp, bneg, sim = _sc_gather_scores(
        users, pos_items, neg_flat, user_embeds, item_embeds, beta_uD,
        beta_iD, nbr_pad, cons_pad)
    return _tc_finish(possc, negsc.reshape(_B, _NN), iisc.reshape(_B, 16),
                      bu, bp, bneg.reshape(_B, _NN), sim, user_embeds,
                      item_embeds)


# speedup vs baseline: 1.0333x; 1.0008x over previous
import jax
import jax.numpy as jnp
from jax import lax
from jax.experimental import pallas as pl
from jax.experimental.pallas import tpu as pltpu
from jax.experimental.pallas import tpu_sc as plsc

_B = 4096
_NN = 200
_K = 10
_D = 64
_W1 = 1e-06
_W2 = 1.0
_W3 = 1e-06
_W4 = 1.0
_NEG_WEIGHT = 300.0
_GAMMA = 1e-4
_LAMBDA = 2.75

_info = plsc.get_sparse_core_info()
_NC, _NS = _info.num_cores, _info.num_subcores
_NW = _NC * _NS
_NB = _B // _NW
_NCH = 128
_NCHUNKS = _NB * _NN // _NCH

def _lane_set(acc, lane, jj, s):
    return jnp.where(lane == jj, s, 0.0 if acc is None else acc)


def _load4(buf, row):
    return [buf[row, pl.ds(t * 16, 16)] for t in range(4)]


def _dot64(buf, row, u4, lane):
    acc = buf[row, pl.ds(0, 16)] * u4[0]
    for t in range(1, 4):
        acc = acc + buf[row, pl.ds(t * 16, 16)] * u4[t]
    for sh in (8, 4, 2, 1):
        idx = jnp.bitwise_xor(lane, sh)
        acc = acc + acc.at[idx].get(mode="promise_in_bounds")
    return acc


def _sc_body(users_h, pos_h, negf_h, uemb_h, iemb_h, buD_h, biD_h, nbr_h,
             cons_h,
             possc_o, negsc_o, iisc_o, buo_o, bpo_o, bnego_o, simo_o,
             uidx_v, pidx_v, nidx_v, u_v, p_v,
             nb0_v, nb1_v, ii0_v, ii1_v, ii2_v, ii3_v,
             nbr_v, sim_v, bu_v, bp_v, bneg_v, possc_v, negsc_v, iisc_v,
             sem_u, sem_p, sem_bu, sem_bp, sem_nbr, sem_sim, sem_bneg,
             sem_n0, sem_n1, sem_i0, sem_i1, sem_i2, sem_i3):
    nbufs = ((nb0_v, sem_n0), (nb1_v, sem_n1))
    iibufs = ((ii0_v, sem_i0), (ii1_v, sem_i1), (ii2_v, sem_i2),
              (ii3_v, sem_i3))
    wid = lax.axis_index("s") * _NC + lax.axis_index("c")
    base = wid * _NB

    pltpu.sync_copy(users_h.at[pl.ds(base, _NB)], uidx_v)
    pltpu.sync_copy(pos_h.at[pl.ds(base, _NB)], pidx_v)
    pltpu.sync_copy(negf_h.at[pl.ds(base * _NN, _NB * _NN)], nidx_v)

    cu = pltpu.make_async_copy(uemb_h.at[uidx_v], u_v, sem_u)
    cu.start()
    cp = pltpu.make_async_copy(iemb_h.at[pidx_v], p_v, sem_p)
    cp.start()
    cbu = pltpu.make_async_copy(buD_h.at[uidx_v], bu_v, sem_bu)
    cbu.start()
    cbp = pltpu.make_async_copy(biD_h.at[pidx_v], bp_v, sem_bp)
    cbp.start()
    cnbr = pltpu.make_async_copy(nbr_h.at[pidx_v], nbr_v, sem_nbr)
    cnbr.start()
    csim = pltpu.make_async_copy(cons_h.at[pidx_v], sim_v, sem_sim)
    csim.start()

    def _fire_neg(c, buf, sem):
        idx = nidx_v.at[pl.ds(c * _NCH, _NCH)]
        pltpu.make_async_copy(iemb_h.at[idx], buf, sem).start()
        pltpu.make_async_copy(biD_h.at[idx],
                              bneg_v.at[pl.ds(c * _NCH, _NCH)],
                              sem_bneg).start()

    for par in range(2):
        _fire_neg(par, nbufs[par][0], nbufs[par][1])

    cu.wait()
    cp.wait()

    lane = lax.iota(jnp.int32, 16)

    def pos_body(gb, carry):
        b0 = gb * 16
        acc = None
        for jj in range(16):
            b = b0 + jj
            u4 = _load4(u_v, b)
            acc = _lane_set(acc, lane, jj, _dot64(p_v, b, u4, lane))
        possc_v[pl.ds(b0, 16)] = acc
        return carry
    lax.fori_loop(0, _NB // 16, pos_body, 0)

    def neg_body(g, carry):
        for par in range(2):
            buf, sem = nbufs[par]
            c = 2 * g + par
            idx = nidx_v.at[pl.ds(c * _NCH, _NCH)]
            pltpu.make_async_copy(iemb_h.at[idx], buf, sem).wait()
            pltpu.make_async_copy(biD_h.at[idx],
                                  bneg_v.at[pl.ds(c * _NCH, _NCH)],
                                  sem_bneg).wait()
            cbase = c * _NCH

            def grp(k, kc):
                r0 = k * 16
                acc = None
                for jj in range(16):
                    b = (cbase + r0 + jj) // _NN
                    u4 = _load4(u_v, b)
                    acc = _lane_set(acc, lane, jj,
                                    _dot64(buf, r0 + jj, u4, lane))
                negsc_v[pl.ds(cbase + r0, 16)] = acc
                return kc
            lax.fori_loop(0, _NCH // 16, grp, 0)

            nc = c + 2

            @pl.when(nc < _NCHUNKS)
            def _():
                _fire_neg(nc, buf, sem)
        return carry
    lax.fori_loop(0, _NCHUNKS // 2, neg_body, 0)

    cnbr.wait()

    def _fire_ii(b, buf, sem):
        pltpu.make_async_copy(iemb_h.at[nbr_v.at[b]], buf, sem).start()

    for par in range(4):
        _fire_ii(par, iibufs[par][0], iibufs[par][1])

    def ii_body(g, carry):
        for par in range(4):
            buf, sem = iibufs[par]
            b = 4 * g + par
            pltpu.make_async_copy(iemb_h.at[nbr_v.at[b]], buf, sem).wait()
            u4 = _load4(u_v, b)
            acc = None
            for j in range(_K):
                acc = _lane_set(acc, lane, j, _dot64(buf, j, u4, lane))
            acc = jnp.where(lane < _K, acc, 0.0)
            iisc_v[pl.ds(b * 16, 16)] = acc

            nb = b + 4

            @pl.when(nb < _NB)
            def _():
                _fire_ii(nb, buf, sem)
        return carry
    lax.fori_loop(0, _NB // 4, ii_body, 0)

    cbu.wait()
    cbp.wait()
    csim.wait()

    pltpu.sync_copy(possc_v, possc_o.at[pl.ds(base, _NB)])
    pltpu.sync_copy(negsc_v, negsc_o.at[pl.ds(base * _NN, _NB * _NN)])
    pltpu.sync_copy(iisc_v, iisc_o.at[pl.ds(base * 16, _NB * 16)])
    pltpu.sync_copy(bu_v, buo_o.at[pl.ds(base, _NB)])
    pltpu.sync_copy(bp_v, bpo_o.at[pl.ds(base, _NB)])
    pltpu.sync_copy(bneg_v, bnego_o.at[pl.ds(base * _NN, _NB * _NN)])
    pltpu.sync_copy(sim_v, simo_o.at[pl.ds(base, _NB)])


def _sc_gather_scores(users, pos_items, neg_flat, user_embeds, item_embeds,
                      beta_uD, beta_iD, ii_neighbor_mat, ii_constraint_mat):
    mesh = plsc.VectorSubcoreMesh(core_axis_name="c", subcore_axis_name="s")
    f32 = jnp.float32
    out_type = [
        jax.ShapeDtypeStruct((_B,), f32),
        jax.ShapeDtypeStruct((_B * _NN,), f32),
        jax.ShapeDtypeStruct((_B * 16,), f32),
        jax.ShapeDtypeStruct((_B,), f32),
        jax.ShapeDtypeStruct((_B,), f32),
        jax.ShapeDtypeStruct((_B * _NN,), f32),
        jax.ShapeDtypeStruct((_B, 16), f32),
    ]
    scratch_types = [
        pltpu.VMEM((_NB,), jnp.int32),
        pltpu.VMEM((_NB,), jnp.int32),
        pltpu.VMEM((_NB * _NN,), jnp.int32),
        pltpu.VMEM((_NB, _D), f32),
        pltpu.VMEM((_NB, _D), f32),
        pltpu.VMEM((_NCH, _D), f32),
        pltpu.VMEM((_NCH, _D), f32),
        pltpu.VMEM((16, _D), f32),
        pltpu.VMEM((16, _D), f32),
        pltpu.VMEM((16, _D), f32),
        pltpu.VMEM((16, _D), f32),
        pltpu.VMEM((_NB, 16), jnp.int32),
        pltpu.VMEM((_NB, 16), f32),
        pltpu.VMEM((_NB,), f32),
        pltpu.VMEM((_NB,), f32),
        pltpu.VMEM((_NB * _NN,), f32),
        pltpu.VMEM((_NB,), f32),
        pltpu.VMEM((_NB * _NN,), f32),
        pltpu.VMEM((_NB * 16,), f32),
    ] + [pltpu.SemaphoreType.DMA] * 13
    run = pl.kernel(_sc_body, mesh=mesh, out_type=out_type,
                    scratch_types=scratch_types,
                    compiler_params=pltpu.CompilerParams(
                        needs_layout_passes=False,
                        use_tc_tiling_on_sc=False))
    return run(users, pos_items, neg_flat, user_embeds, item_embeds,
               beta_uD, beta_iD, ii_neighbor_mat, ii_constraint_mat)


_RBLK = 2000
_GSTEPS = 100000 // _RBLK


def _tc_body(possc_r, negsc_r, iisc_r, bu_r, bp_r, bneg_r, sim_r, ue_r, ie_r,
             out_r):
    i = pl.program_id(0)

    @pl.when(i == 0)
    def _():
        bu = bu_r[...]
        pos_w = _W1 + _W2 * bu * bp_r[...]
        pos_loss = jnp.sum(pos_w * jax.nn.softplus(-possc_r[...]))
        neg_w = _W3 + _W4 * bu[:, None] * bneg_r[...]
        neg_loss = jnp.sum(neg_w * jax.nn.softplus(negsc_r[...]))
        ii_loss = jnp.sum(sim_r[...][:, :_K]
                          * jax.nn.softplus(-iisc_r[...][:, :_K]))
        out_r[0] = (pos_loss + neg_loss * (_NEG_WEIGHT / _NN)
                    + ii_loss * _LAMBDA)

    nrm = jnp.sum(ue_r[...] * ue_r[...]) + jnp.sum(ie_r[...] * ie_r[...])
    out_r[0] += (0.5 * _GAMMA) * nrm


def _tc_finish(possc, negsc, iisc, bu, bp, bneg, sim, user_embeds,
               item_embeds):
    full = lambda arr: pl.BlockSpec(arr.shape, lambda i: (0,) * arr.ndim)
    grid_spec = pl.GridSpec(
        grid=(_GSTEPS,),
        in_specs=[
            full(possc), full(negsc), full(iisc), full(bu), full(bp),
            full(bneg), full(sim),
            pl.BlockSpec((_RBLK, _D), lambda i: (i, 0)),
            pl.BlockSpec((_RBLK, _D), lambda i: (i, 0)),
        ],
        out_specs=pl.BlockSpec(memory_space=pltpu.SMEM),
    )
    out = pl.pallas_call(
        _tc_body,
        grid_spec=grid_spec,
        out_shape=jax.ShapeDtypeStruct((1,), jnp.float32),
        compiler_params=pltpu.CompilerParams(
            dimension_semantics=("arbitrary",)),
    )(possc, negsc, iisc, bu, bp, bneg, sim, user_embeds, item_embeds)
    return out[0]


def kernel(users, pos_items, neg_items, user_embeds, item_embeds, beta_uD,
           beta_iD, ii_neighbor_mat, ii_constraint_mat):
    neg_flat = neg_items.reshape(-1)
    nbr_pad = jnp.pad(ii_neighbor_mat, ((0, 0), (0, 16 - _K)))
    cons_pad = jnp.pad(ii_constraint_mat, ((0, 0), (0, 16 - _K)))
    possc, negsc, iisc, bu, bp, bneg, sim = _sc_gather_scores(
        users, pos_items, neg_flat, user_embeds, item_embeds, beta_uD,
        beta_iD, nbr_pad, cons_pad)
    return _tc_finish(possc, negsc.reshape(_B, _NN), iisc.reshape(_B, 16),
                      bu, bp, bneg.reshape(_B, _NN), sim, user_embeds,
                      item_embeds)
